# baseline (speedup 1.0000x reference)
import functools

import jax
import jax.numpy as jnp
import numpy as np
from jax.experimental import pallas as pl
from jax.experimental.pallas import tpu as pltpu

_N = 468
_NPAD = 512
_G = 16
_KW = 96
_STARTS = tuple(min(max(64 * t - 16, 0), _NPAD - _KW)
                for t in range(_NPAD // 64))


def _adj_np(n: int, npad: int):
    src, dst = [], []
    for i in range(n):
        for j in range(i + 1, min(i + 10, n)):
            src += [i, j]
            dst += [j, i]
    src = np.concatenate([np.array(src, np.int64), np.arange(n)])
    dst = np.concatenate([np.array(dst, np.int64), np.arange(n)])
    deg = np.zeros((n,), np.float64)
    np.add.at(deg, dst, 1.0)
    dinv = np.where(deg > 0, deg ** -0.5, 0.0)
    a01 = np.zeros((npad, npad), np.float32)
    a01[dst, src] = 1.0
    dinv_pad = np.zeros((npad,), np.float32)
    dinv_pad[:n] = dinv.astype(np.float32)
    nt = npad // 64
    blocks = np.zeros((nt, 64, _KW), np.float32)
    for t, s in enumerate(_STARTS):
        blocks[t] = a01[64 * t:64 * (t + 1), s:s + _KW]
    return a01, blocks, _STARTS, dinv_pad


def _gather_np(npad: int):
    ek = np.zeros((3, 3 * npad, npad), np.float32)
    for k in range(3):
        for node in range(npad):
            ek[k, 3 * node + k, node] = 1.0
    sel = np.zeros((_G, 3 * _G, 3), np.float32)
    for g in range(_G):
        for k in range(3):
            sel[g, k * _G + g, k] = 1.0
    return ek, sel


def _gnn_body(coords_ref, we_ref, be_ref, w1_ref, b1_ref, w2_ref, b2_ref,
              w3_ref, b3_ref, wf_ref, bf_ref, wc_ref, bc_ref,
              ablk_ref, ek_ref, sel_ref, dinv_ref, mask_ref,
              pooled_ref, feats_ref, out_ref, acc_ref):
    mask = mask_ref[...]
    dinv = dinv_ref[...]

    def msg(h, b):
        c = h.shape[1]
        u = h * dinv_t
        u_hi = u.astype(jnp.bfloat16)
        u_lo = (u - u_hi.astype(jnp.float32)).astype(jnp.bfloat16)
        u2 = jnp.concatenate([u_hi, u_lo], axis=1)
        parts = []
        for g in range(_G):
            tiles = []
            for t in range(_NPAD // 64):
                s = g * _NPAD + _STARTS[t]
                v2 = jnp.dot(ablk_ref[t], u2[s:s + _KW],
                             preferred_element_type=jnp.float32)
                tiles.append(v2[:, :c] + v2[:, c:])
            v = jnp.concatenate(tiles, axis=0)
            parts.append(jax.nn.relu(v * dinv + b))
        return jnp.concatenate(parts, axis=0)

    cflat = coords_ref[...]
    c_hi = cflat.astype(jnp.bfloat16)
    s_ks = [jnp.dot(c_hi, ek_ref[k], preferred_element_type=jnp.float32)
            for k in range(3)]
    shi = jnp.concatenate(s_ks, axis=0)
    st_hi = shi.astype(jnp.bfloat16).T
    dinv_t = jnp.concatenate([dinv] * _G, axis=0)
    we = we_ref[...]
    xs = []
    for g in range(_G):
        swh = jnp.dot(sel_ref[g], we,
                      preferred_element_type=jnp.float32)
        xs.append(jnp.dot(st_hi, swh,
                          preferred_element_type=jnp.float32))
    x = jnp.concatenate(xs, axis=0)
    x = x + be_ref[...]
    h = jnp.dot(x, w1_ref[...], preferred_element_type=jnp.float32)
    x = msg(h, b1_ref[...])
    h = jnp.dot(x, w2_ref[...], preferred_element_type=jnp.float32)
    x = msg(h, b2_ref[...])
    h = jnp.dot(x, w3_ref[...], preferred_element_type=jnp.float32)
    x = msg(h, b3_ref[...])
    step = pl.program_id(0)
    rows = []
    for g in range(_G):
        row = jnp.sum(x[g * _NPAD:(g + 1) * _NPAD] * mask,
                      axis=0, keepdims=True)
        pooled_ref[g] = row
        rows.append(row)
    acc_ref[pl.ds(step * _G, _G), :] = jnp.concatenate(rows, axis=0)

    @pl.when(step == pl.num_programs(0) - 1)
    def _head():
        f = jax.nn.relu(
            jnp.dot(acc_ref[...], wf_ref[...],
                    preferred_element_type=jnp.float32) + bf_ref[...])
        feats_ref[...] = f
        out_ref[...] = jnp.dot(f, wc_ref[...],
                               preferred_element_type=jnp.float32) + bc_ref[...]


@functools.partial(jax.jit, static_argnames=())
def kernel(landmarks, We, be, W1, b1, W2, b2, W3, b3, Wf, bf, Wc, bc):
    bsz = landmarks.shape[0]
    n = landmarks.shape[1] // 3

    coords = jnp.pad(landmarks, ((0, 0), (0, 3 * _NPAD - landmarks.shape[1])))

    _, ablk_np, _, dinv_np = _adj_np(n, _NPAD)
    ablk = jnp.asarray(ablk_np, dtype=jnp.bfloat16)
    ek_np, sel_np = _gather_np(_NPAD)
    ek = jnp.asarray(ek_np, dtype=jnp.bfloat16)
    sel = jnp.asarray(sel_np)
    dinv = jnp.asarray(dinv_np[:, None])
    mask = jnp.asarray(
        (np.arange(_NPAD) < n).astype(np.float32)[:, None] / n)

    c1 = W1.shape[1]
    c3 = W3.shape[1]
    cf = Wf.shape[1]
    cc = Wc.shape[1]
    const = pl.BlockSpec(None, lambda b: (0, 0))
    const3 = pl.BlockSpec(None, lambda b: (0, 0, 0))

    pooled, feats, out = pl.pallas_call(
        _gnn_body,
        grid=(bsz // _G,),
        in_specs=[
            pl.BlockSpec((_G, 3 * _NPAD), lambda b: (b, 0)),
            const, const, const, const, const, const, const, const,
            const, const, const, const,
            const3, const3, const3, const, const,
        ],
        out_specs=[
            pl.BlockSpec((_G, 1, c1), lambda b: (b, 0, 0)),
            pl.BlockSpec((bsz, cf), lambda b: (0, 0)),
            pl.BlockSpec((bsz, cc), lambda b: (0, 0)),
        ],
        out_shape=[
            jax.ShapeDtypeStruct((bsz, 1, c1), jnp.float32),
            jax.ShapeDtypeStruct((bsz, cf), jnp.float32),
            jax.ShapeDtypeStruct((bsz, cc), jnp.float32),
        ],
        scratch_shapes=[pltpu.VMEM((bsz, c3), jnp.float32)],
    )(coords, We, be.reshape(1, -1), W1, b1.reshape(1, -1),
      W2, b2.reshape(1, -1), W3, b3.reshape(1, -1),
      Wf, bf.reshape(1, -1), Wc, bc.reshape(1, -1),
      ablk, ek, sel, dinv, mask)

    return (out.reshape(bsz, 1, cc),
            feats.reshape(bsz, 1, cf),
            pooled)


# speedup vs baseline: 1.0194x; 1.0194x over previous
import functools

import jax
import jax.numpy as jnp
import numpy as np
from jax.experimental import pallas as pl
from jax.experimental.pallas import tpu as pltpu

_N = 468
_NPAD = 512
_G = 32
_KW = 96
_STARTS = tuple(min(max(64 * t - 16, 0), _NPAD - _KW)
                for t in range(_NPAD // 64))


def _adj_np(n: int, npad: int):
    src, dst = [], []
    for i in range(n):
        for j in range(i + 1, min(i + 10, n)):
            src += [i, j]
            dst += [j, i]
    src = np.concatenate([np.array(src, np.int64), np.arange(n)])
    dst = np.concatenate([np.array(dst, np.int64), np.arange(n)])
    deg = np.zeros((n,), np.float64)
    np.add.at(deg, dst, 1.0)
    dinv = np.where(deg > 0, deg ** -0.5, 0.0)
    a01 = np.zeros((npad, npad), np.float32)
    a01[dst, src] = 1.0
    dinv_pad = np.zeros((npad,), np.float32)
    dinv_pad[:n] = dinv.astype(np.float32)
    nt = npad // 64
    blocks = np.zeros((nt, 64, _KW), np.float32)
    for t, s in enumerate(_STARTS):
        blocks[t] = a01[64 * t:64 * (t + 1), s:s + _KW]
    return a01, blocks, _STARTS, dinv_pad


def _gather_np(npad: int):
    ek = np.zeros((3, 3 * npad, npad), np.float32)
    for k in range(3):
        for node in range(npad):
            ek[k, 3 * node + k, node] = 1.0
    sel = np.zeros((_G, 3 * _G, 3), np.float32)
    for g in range(_G):
        for k in range(3):
            sel[g, k * _G + g, k] = 1.0
    return ek, sel


def _gnn_body(coords_ref, we_ref, be_ref, w1_ref, b1_ref, w2_ref, b2_ref,
              w3_ref, b3_ref, wf_ref, bf_ref, wc_ref, bc_ref,
              ablk_ref, ek_ref, sel_ref, dinv_ref, mask_ref,
              pooled_ref, feats_ref, out_ref, acc_ref):
    mask = mask_ref[...]
    dinv = dinv_ref[...]

    def msg(h, b):
        c = h.shape[1]
        u = h * dinv_t
        u_hi = u.astype(jnp.bfloat16)
        u_lo = (u - u_hi.astype(jnp.float32)).astype(jnp.bfloat16)
        u2 = jnp.concatenate([u_hi, u_lo], axis=1)
        parts = []
        for g in range(_G):
            tiles = []
            for t in range(_NPAD // 64):
                s = g * _NPAD + _STARTS[t]
                v2 = jnp.dot(ablk_ref[t], u2[s:s + _KW],
                             preferred_element_type=jnp.float32)
                tiles.append(v2[:, :c] + v2[:, c:])
            v = jnp.concatenate(tiles, axis=0)
            parts.append(jax.nn.relu(v * dinv + b))
        return jnp.concatenate(parts, axis=0)

    cflat = coords_ref[...]
    c_hi = cflat.astype(jnp.bfloat16)
    s_ks = [jnp.dot(c_hi, ek_ref[k], preferred_element_type=jnp.float32)
            for k in range(3)]
    shi = jnp.concatenate(s_ks, axis=0)
    st_hi = shi.astype(jnp.bfloat16).T
    dinv_t = jnp.concatenate([dinv] * _G, axis=0)
    we = we_ref[...]
    xs = []
    for g in range(_G):
        swh = jnp.dot(sel_ref[g], we,
                      preferred_element_type=jnp.float32)
        xs.append(jnp.dot(st_hi, swh,
                          preferred_element_type=jnp.float32))
    x = jnp.concatenate(xs, axis=0)
    x = x + be_ref[...]
    h = jnp.dot(x, w1_ref[...], preferred_element_type=jnp.float32)
    x = msg(h, b1_ref[...])
    h = jnp.dot(x, w2_ref[...], preferred_element_type=jnp.float32)
    x = msg(h, b2_ref[...])
    h = jnp.dot(x, w3_ref[...], preferred_element_type=jnp.float32)
    x = msg(h, b3_ref[...])
    step = pl.program_id(0)
    rows = []
    for g in range(_G):
        row = jnp.sum(x[g * _NPAD:(g + 1) * _NPAD] * mask,
                      axis=0, keepdims=True)
        pooled_ref[g] = row
        rows.append(row)
    acc_ref[pl.ds(step * _G, _G), :] = jnp.concatenate(rows, axis=0)

    @pl.when(step == pl.num_programs(0) - 1)
    def _head():
        f = jax.nn.relu(
            jnp.dot(acc_ref[...], wf_ref[...],
                    preferred_element_type=jnp.float32) + bf_ref[...])
        feats_ref[...] = f
        out_ref[...] = jnp.dot(f, wc_ref[...],
                               preferred_element_type=jnp.float32) + bc_ref[...]


@functools.partial(jax.jit, static_argnames=())
def kernel(landmarks, We, be, W1, b1, W2, b2, W3, b3, Wf, bf, Wc, bc):
    bsz = landmarks.shape[0]
    n = landmarks.shape[1] // 3

    coords = jnp.pad(landmarks, ((0, 0), (0, 3 * _NPAD - landmarks.shape[1])))

    _, ablk_np, _, dinv_np = _adj_np(n, _NPAD)
    ablk = jnp.asarray(ablk_np, dtype=jnp.bfloat16)
    ek_np, sel_np = _gather_np(_NPAD)
    ek = jnp.asarray(ek_np, dtype=jnp.bfloat16)
    sel = jnp.asarray(sel_np)
    dinv = jnp.asarray(dinv_np[:, None])
    mask = jnp.asarray(
        (np.arange(_NPAD) < n).astype(np.float32)[:, None] / n)

    c1 = W1.shape[1]
    c3 = W3.shape[1]
    cf = Wf.shape[1]
    cc = Wc.shape[1]
    const = pl.BlockSpec(None, lambda b: (0, 0))
    const3 = pl.BlockSpec(None, lambda b: (0, 0, 0))

    pooled, feats, out = pl.pallas_call(
        _gnn_body,
        grid=(bsz // _G,),
        in_specs=[
            pl.BlockSpec((_G, 3 * _NPAD), lambda b: (b, 0)),
            const, const, const, const, const, const, const, const,
            const, const, const, const,
            const3, const3, const3, const, const,
        ],
        out_specs=[
            pl.BlockSpec((_G, 1, c1), lambda b: (b, 0, 0)),
            pl.BlockSpec((bsz, cf), lambda b: (0, 0)),
            pl.BlockSpec((bsz, cc), lambda b: (0, 0)),
        ],
        out_shape=[
            jax.ShapeDtypeStruct((bsz, 1, c1), jnp.float32),
            jax.ShapeDtypeStruct((bsz, cf), jnp.float32),
            jax.ShapeDtypeStruct((bsz, cc), jnp.float32),
        ],
        scratch_shapes=[pltpu.VMEM((bsz, c3), jnp.float32)],
    )(coords, We, be.reshape(1, -1), W1, b1.reshape(1, -1),
      W2, b2.reshape(1, -1), W3, b3.reshape(1, -1),
      Wf, bf.reshape(1, -1), Wc, bc.reshape(1, -1),
      ablk, ek, sel, dinv, mask)

    return (out.reshape(bsz, 1, cc),
            feats.reshape(bsz, 1, cf),
            pooled)
